# baseline (speedup 1.0000x reference)
import functools

import jax
import jax.numpy as jnp
from jax.experimental import pallas as pl
from jax.experimental.pallas import tpu as pltpu
from jax.experimental.pallas import tpu_sc as plsc

N = 10000
E = 160000
H = 500
HP = 512
G4 = 4 * HP
HEADS = 5
ZP = HEADS * HP
NEG_SLOPE = 0.2

BLK = 256
NPAD = 10240
NBLK = NPAD // BLK
EPAD = E + BLK
CC = 1024
CTRL_LEN = ((E + CC - 1) // CC) * CC
GW = 128

_VMEM = pltpu.MemorySpace.VMEM
_HBM = pltpu.MemorySpace.HBM
_SMEM = pltpu.MemorySpace.SMEM


def _sc_mesh():
    return plsc.VectorSubcoreMesh(core_axis_name="core",
                                  subcore_axis_name="subcore")


def _sc_gather(table, idx):
    orig_dtype = table.dtype
    if orig_dtype == jnp.bfloat16:
        r, w = table.shape
        table = jax.lax.bitcast_convert_type(
            table.reshape(r, w // 2, 2), jnp.uint32)
    width = table.shape[1]
    factor = width // 128
    if factor > 1:
        table = table.reshape(table.shape[0] * factor, 128)
        idx = (idx[:, None] * factor
               + jnp.arange(factor, dtype=jnp.int32)[None, :]).reshape(-1)
        width = 128
    rows = idx.shape[0]
    idx2 = idx.reshape(1, rows)

    @functools.partial(
        pl.kernel,
        out_type=jax.ShapeDtypeStruct((rows, width), table.dtype),
        mesh=_sc_mesh(),
    )
    def k(x_hbm, i_hbm, o_hbm):
        def body(i_vmem, o_vmem):
            pltpu.sync_copy(x_hbm.at[i_vmem.at[0]], o_vmem)

        pltpu.emit_pipeline(
            body,
            grid=(rows // GW,),
            in_specs=[pl.BlockSpec((1, GW), lambda i: (0, i))],
            out_specs=[pl.BlockSpec((GW, width), lambda i: (i, 0))],
            core_axis_name=("core", "subcore"),
            dimension_semantics=(pltpu.PARALLEL,),
        )(i_hbm, o_hbm)

    out = k(table, idx2)
    if factor > 1:
        out = out.reshape(rows // factor, 128 * factor)
    if orig_dtype == jnp.bfloat16:
        out = jax.lax.bitcast_convert_type(out, jnp.bfloat16)
        out = out.reshape(out.shape[0], out.shape[1] * 2)
    return out


def _embed_kernel(tok_ref, emb_ref, o_ref):
    tok = tok_ref[...]
    lanes = jax.lax.broadcasted_iota(jnp.int32, (BLK, 128), 1)
    oh = (tok == lanes).astype(jnp.bfloat16)
    o_ref[...] = jnp.dot(oh, emb_ref[...],
                         preferred_element_type=jnp.float32
                         ).astype(jnp.bfloat16)


def _embed(tokens_pad, emb_pad):
    return pl.pallas_call(
        _embed_kernel,
        grid=(NBLK,),
        in_specs=[
            pl.BlockSpec((BLK, 1), lambda b: (b, 0)),
            pl.BlockSpec((128, HP), lambda b: (0, 0)),
        ],
        out_specs=pl.BlockSpec((BLK, HP), lambda b: (b, 0)),
        out_shape=jax.ShapeDtypeStruct((NPAD, HP), jnp.bfloat16),
    )(tokens_pad, emb_pad)


def _lstm_kernel(xe_hbm, k_hbm, dmax_ref, wih_ref, whh_ref, b_ref,
                 h_ref, c_ref, xbuf, ksm, dsem, ksem):
    h_ref[...] = jnp.zeros((NPAD, HP), jnp.float32)
    c_ref[...] = jnp.zeros((NPAD, HP), jnp.float32)
    dmax = dmax_ref[0]
    nchunks = (dmax + CC - 1) // CC

    def chunk_body(ci, off):
        cp = pltpu.make_async_copy(k_hbm.at[pl.ds(ci * CC, CC)], ksm, ksem)
        cp.start()
        cp.wait()
        tmax = jnp.minimum(CC, dmax - ci * CC)

        def step(tt, off):
            kt = ksm[tt]
            nblk = (kt + BLK - 1) // BLK

            def blk(j, _):
                base = pl.multiple_of(off + j * BLK, 8)
                jb = pl.multiple_of(j * BLK, BLK)
                cpx = pltpu.make_async_copy(
                    xe_hbm.at[pl.ds(base, BLK), :], xbuf, dsem)
                cpx.start()
                cpx.wait()
                hb = h_ref[pl.ds(jb, BLK), :]
                cb = c_ref[pl.ds(jb, BLK), :]
                gates = (
                    jnp.dot(xbuf[...], wih_ref[...],
                            preferred_element_type=jnp.float32)
                    + jnp.dot(hb.astype(jnp.bfloat16), whh_ref[...],
                              preferred_element_type=jnp.float32)
                    + b_ref[0:1, :]
                )
                gi = jax.nn.sigmoid(gates[:, 0:HP])
                gf = jax.nn.sigmoid(gates[:, HP:2 * HP])
                gg = jnp.tanh(gates[:, 2 * HP:3 * HP])
                go = jax.nn.sigmoid(gates[:, 3 * HP:4 * HP])
                c_new = gf * cb + gi * gg
                h_new = go * jnp.tanh(c_new)
                rows = jax.lax.broadcasted_iota(jnp.int32, (BLK, 1), 0)
                act = (rows + j * BLK) < kt
                h_ref[pl.ds(jb, BLK), :] = jnp.where(act, h_new, hb)
                c_ref[pl.ds(jb, BLK), :] = jnp.where(act, c_new, cb)
                return 0

            jax.lax.fori_loop(0, nblk, blk, 0)
            return off + kt

        return jax.lax.fori_loop(0, tmax, step, off)

    jax.lax.fori_loop(0, nchunks, chunk_body, jnp.int32(0))


def _lstm(xe, karr, dmax_arr, wih_t, whh_t, bias):
    return pl.pallas_call(
        _lstm_kernel,
        in_specs=[
            pl.BlockSpec(memory_space=_HBM),
            pl.BlockSpec(memory_space=_HBM),
            pl.BlockSpec(memory_space=_SMEM),
            pl.BlockSpec(memory_space=_VMEM),
            pl.BlockSpec(memory_space=_VMEM),
            pl.BlockSpec(memory_space=_VMEM),
        ],
        out_specs=pl.BlockSpec(memory_space=_VMEM),
        out_shape=jax.ShapeDtypeStruct((NPAD, HP), jnp.float32),
        scratch_shapes=[
            pltpu.VMEM((NPAD, HP), jnp.float32),
            pltpu.VMEM((BLK, HP), jnp.bfloat16),
            pltpu.SMEM((CC,), jnp.int32),
            pltpu.SemaphoreType.DMA,
            pltpu.SemaphoreType.DMA,
        ],
        compiler_params=pltpu.CompilerParams(
            vmem_limit_bytes=60 * 1024 * 1024),
    )(xe, karr, dmax_arr, wih_t, whh_t, bias)


def _combine_kernel(x_ref, hn_ref, ws_ref, wn_ref, b_ref, o_ref):
    o_ref[...] = jax.nn.relu(
        jnp.dot(x_ref[...], ws_ref[...], preferred_element_type=jnp.float32)
        + jnp.dot(hn_ref[...].astype(jnp.bfloat16), wn_ref[...],
                  preferred_element_type=jnp.float32)
        + b_ref[0:1, :]
    ).astype(jnp.bfloat16)


def _combine(x, hn, ws_t, wn_t, bias):
    return pl.pallas_call(
        _combine_kernel,
        grid=(NBLK,),
        in_specs=[
            pl.BlockSpec((BLK, HP), lambda b: (b, 0)),
            pl.BlockSpec((BLK, HP), lambda b: (b, 0)),
            pl.BlockSpec((HP, HP), lambda b: (0, 0)),
            pl.BlockSpec((HP, HP), lambda b: (0, 0)),
            pl.BlockSpec((8, HP), lambda b: (0, 0)),
        ],
        out_specs=pl.BlockSpec((BLK, HP), lambda b: (b, 0)),
        out_shape=jax.ShapeDtypeStruct((NPAD, HP), jnp.bfloat16),
    )(x, hn, ws_t, wn_t, bias)


def _eler_kernel(x_ref, w_ref, o_ref):
    o_ref[...] = jnp.dot(x_ref[...], w_ref[...],
                         preferred_element_type=jnp.float32)


def _eler(x, w_eler):
    return pl.pallas_call(
        _eler_kernel,
        grid=(NBLK,),
        in_specs=[
            pl.BlockSpec((BLK, HP), lambda b: (b, 0)),
            pl.BlockSpec((HP, 128), lambda b: (0, 0)),
        ],
        out_specs=pl.BlockSpec((BLK, 128), lambda b: (b, 0)),
        out_shape=jax.ShapeDtypeStruct((NPAD, 128), jnp.float32),
    )(x, w_eler)


def _gat_kernel(sb_ref, xe_hbm, els_hbm, dst_hbm, er_ref, gw_ref, gb_ref,
                wr_ref, o_ref, acc_ref, xebuf, elbuf, dstbuf, sem1, sem2,
                sem3):
    b = pl.program_id(0)

    @pl.when(b == 0)
    def _():
        o_ref[...] = jnp.zeros((8, 128), jnp.float32)

    e_begin = sb_ref[b]
    e_end = sb_ref[b + 1]
    nchunks = (e_end - e_begin + BLK - 1) // BLK
    base_node = b * BLK
    lanes = jax.lax.broadcasted_iota(jnp.int32, (BLK, BLK), 1)

    er_rows = []
    for h in range(HEADS):
        col = er_ref[:, 16 + h:16 + h + 1]
        er_rows.append(jnp.transpose(col, (1, 0)))

    def load_meta(k):
        eb = pl.multiple_of(e_begin + k * BLK, 8)
        cp1 = pltpu.make_async_copy(
            els_hbm.at[pl.ds(eb, BLK), :], elbuf, sem1)
        cp2 = pltpu.make_async_copy(
            dst_hbm.at[pl.ds(eb, BLK), :], dstbuf, sem2)
        cp1.start()
        cp2.start()
        cp1.wait()
        cp2.wait()
        valid = (dstbuf[...] - base_node) == lanes
        return valid

    def emat(h, valid):
        e = elbuf[:, h:h + 1] + er_rows[h]
        e = jnp.where(e >= 0, e, NEG_SLOPE * e)
        return e, valid

    def pass1(k, m):
        valid = load_meta(k)
        out = []
        for h in range(HEADS):
            e, v = emat(h, valid)
            e = jnp.where(v, e, -jnp.inf)
            mh = jnp.max(e, axis=0, keepdims=True)
            out.append(jnp.maximum(m[h], mh))
        return tuple(out)

    m0 = tuple(jnp.full((1, BLK), -jnp.inf, jnp.float32)
               for _ in range(HEADS))
    m = jax.lax.fori_loop(0, nchunks, pass1, m0)
    mfin = [jnp.where(jnp.isinf(mh), 0.0, mh) for mh in m]

    acc_ref[...] = jnp.zeros((BLK, ZP), jnp.float32)

    def pass2(k, s):
        valid = load_meta(k)
        cpx = pltpu.make_async_copy(
            xe_hbm.at[pl.ds(pl.multiple_of(e_begin + k * BLK, 8), BLK), :],
            xebuf, sem3)
        cpx.start()
        cpx.wait()
        ze = jnp.dot(xebuf[...], gw_ref[...],
                     preferred_element_type=jnp.float32
                     ).astype(jnp.bfloat16)
        out = []
        for h in range(HEADS):
            e, v = emat(h, valid)
            w = jnp.where(v, jnp.exp(e - mfin[h]), 0.0)
            sh = jnp.sum(w, axis=0, keepdims=True)
            out.append(s[h] + sh)
            contrib = jax.lax.dot_general(
                w.astype(jnp.bfloat16), ze[:, h * HP:(h + 1) * HP],
                (((0,), (0,)), ((), ())),
                preferred_element_type=jnp.float32)
            acc_ref[:, h * HP:(h + 1) * HP] += contrib
        return tuple(out)

    s0 = tuple(jnp.zeros((1, BLK), jnp.float32) for _ in range(HEADS))
    s = jax.lax.fori_loop(0, nchunks, pass2, s0)

    xm = jnp.zeros((BLK, HP), jnp.float32)
    for h in range(HEADS):
        sT = jnp.transpose(s[h], (1, 0))
        outh = acc_ref[:, h * HP:(h + 1) * HP] / jnp.maximum(sT, 1e-9)
        xm = xm + jax.nn.relu(outh + gb_ref[h:h + 1, :])
    xm = xm * (1.0 / HEADS)
    xm = jnp.where(xm > 0, xm, jnp.exp(xm) - 1.0)
    scores = jnp.dot(xm.astype(jnp.bfloat16), wr_ref[...],
                     preferred_element_type=jnp.float32)
    rows = jax.lax.broadcasted_iota(jnp.int32, (BLK, 128), 0)
    scores = jnp.where(rows + base_node < N, scores, 0.0)
    part = jnp.sum(scores, axis=0, keepdims=True)
    o_ref[...] += jnp.broadcast_to(part, (8, 128))


def _gat(sb, xe_gat, elsrc, dst_sp, er_n, gw_pad, gb_pad, wr_pad):
    return pl.pallas_call(
        _gat_kernel,
        grid=(NBLK,),
        in_specs=[
            pl.BlockSpec(memory_space=_SMEM),
            pl.BlockSpec(memory_space=_HBM),
            pl.BlockSpec(memory_space=_HBM),
            pl.BlockSpec(memory_space=_HBM),
            pl.BlockSpec((BLK, 128), lambda b: (b, 0)),
            pl.BlockSpec((HP, ZP), lambda b: (0, 0)),
            pl.BlockSpec((8, HP), lambda b: (0, 0)),
            pl.BlockSpec((HP, 128), lambda b: (0, 0)),
        ],
        out_specs=pl.BlockSpec((8, 128), lambda b: (0, 0)),
        out_shape=jax.ShapeDtypeStruct((8, 128), jnp.float32),
        scratch_shapes=[
            pltpu.VMEM((BLK, ZP), jnp.float32),
            pltpu.VMEM((BLK, HP), jnp.bfloat16),
            pltpu.VMEM((BLK, 128), jnp.float32),
            pltpu.VMEM((BLK, 1), jnp.int32),
            pltpu.SemaphoreType.DMA,
            pltpu.SemaphoreType.DMA,
            pltpu.SemaphoreType.DMA,
        ],
        compiler_params=pltpu.CompilerParams(
            vmem_limit_bytes=60 * 1024 * 1024),
    )(sb, xe_gat, elsrc, dst_sp, er_n, gw_pad, gb_pad, wr_pad)


def _pad_gate_w(w):
    out = jnp.zeros((HP, G4), jnp.float32)
    for g in range(4):
        out = out.at[:H, g * HP:g * HP + H].set(
            w[g * H:(g + 1) * H, :].T)
    return out


def _pad_gate_b(b):
    out = jnp.zeros((8, G4), jnp.float32)
    for g in range(4):
        out = out.at[:, g * HP:g * HP + H].set(
            jnp.broadcast_to(b[g * H:(g + 1) * H][None, :], (8, H)))
    return out


def kernel(node_tokens, edge_index, emb, sage_Wih, sage_Whh, sage_bih,
           sage_bhh, sage_selfW, sage_selfb, sage_neighW, sage_neighb,
           gat_W, gat_attn_l, gat_attn_r, gat_b, readout_W, readout_b):
    f32 = jnp.float32
    src = edge_index[0].astype(jnp.int32)
    dst = edge_index[1].astype(jnp.int32)
    order = jnp.argsort(dst)
    src_s = src[order]
    dst_s = dst[order]
    deg = jnp.bincount(dst, length=N).astype(jnp.int32)
    starts = (jnp.cumsum(deg) - deg).astype(jnp.int32)
    dmax = jnp.max(deg)

    perm = jnp.argsort(-deg).astype(jnp.int32)
    rank = jnp.zeros((N,), jnp.int32).at[perm].set(
        jnp.arange(N, dtype=jnp.int32))
    cnt = jnp.bincount(deg, length=E + 1)
    le = jnp.cumsum(cnt)
    karr = (N - le)[:E].astype(jnp.int32)
    karr = jnp.concatenate(
        [karr, jnp.zeros((CTRL_LEN - E,), jnp.int32)])
    off = jnp.concatenate(
        [jnp.zeros((1,), jnp.int32),
         jnp.cumsum(karr[:E - 1], dtype=jnp.int32)])
    t_e = jnp.arange(E, dtype=jnp.int32) - starts[dst_s]
    p_e = off[t_e] + rank[dst_s]
    gsrc = jnp.zeros((EPAD,), jnp.int32).at[p_e].set(src_s)
    dmax_arr = jnp.reshape(dmax, (1,)).astype(jnp.int32)

    rank_pad = jnp.concatenate(
        [rank, jnp.zeros((NPAD - N,), jnp.int32)])
    src_pad = jnp.concatenate(
        [src_s, jnp.zeros((EPAD - E,), jnp.int32)])
    dst_sp = jnp.concatenate(
        [dst_s, jnp.full((EPAD - E,), -1, jnp.int32)]).reshape(EPAD, 1)
    sb = jnp.concatenate(
        [starts[::BLK], jnp.full((1,), E, jnp.int32),
         jnp.zeros((7,), jnp.int32)])

    tokens_pad = jnp.concatenate(
        [node_tokens.astype(jnp.int32),
         jnp.zeros((NPAD - N,), jnp.int32)]).reshape(NPAD, 1)
    emb_pad = jnp.zeros((128, HP), f32).at[:119, :H].set(emb).astype(jnp.bfloat16)

    wih_t = [_pad_gate_w(sage_Wih[i]).astype(jnp.bfloat16) for i in range(3)]
    whh_t = [_pad_gate_w(sage_Whh[i]).astype(jnp.bfloat16) for i in range(3)]
    bgate = [_pad_gate_b(sage_bih[i] + sage_bhh[i]) for i in range(3)]
    ws_t = [jnp.zeros((HP, HP), f32).at[:H, :H].set(
        sage_selfW[i].T).astype(jnp.bfloat16) for i in range(3)]
    wn_t = [jnp.zeros((HP, HP), f32).at[:H, :H].set(
        sage_neighW[i].T).astype(jnp.bfloat16) for i in range(3)]
    bcomb = [jnp.zeros((8, HP), f32).at[:, :H].set(
        jnp.broadcast_to((sage_selfb[i] + sage_neighb[i])[None, :], (8, H)))
        for i in range(3)]

    gat_w3 = gat_W.reshape(H, HEADS, H)
    gw_pad = jnp.zeros((HP, ZP), f32)
    for h in range(HEADS):
        gw_pad = gw_pad.at[:H, h * HP:h * HP + H].set(gat_w3[:, h, :])
    gw_pad = gw_pad.astype(jnp.bfloat16)
    w_el = jnp.einsum('ihk,hk->ih', gat_w3, gat_attn_l)
    w_er = jnp.einsum('ihk,hk->ih', gat_w3, gat_attn_r)
    w_eler = jnp.zeros((HP, 128), f32)
    w_eler = w_eler.at[:H, 0:HEADS].set(w_el)
    w_eler = w_eler.at[:H, 16:16 + HEADS].set(w_er)
    w_eler = w_eler.astype(jnp.bfloat16)
    gb_pad = jnp.zeros((8, HP), f32).at[:HEADS, :H].set(
        gat_b.reshape(HEADS, H))
    wr_pad = jnp.zeros((HP, 128), f32).at[:H, 0:1].set(
        readout_W).astype(jnp.bfloat16)

    x = _embed(tokens_pad, emb_pad)
    for i in range(3):
        xe = _sc_gather(x, gsrc)
        h_p = _lstm(xe, karr, dmax_arr, wih_t[i], whh_t[i], bgate[i])
        h_n = _sc_gather(h_p, rank_pad)
        x = _combine(x, h_n, ws_t[i], wn_t[i], bcomb[i])

    eler = _eler(x, w_eler)
    xe_gat = _sc_gather(x, src_pad)
    elsrc = _sc_gather(eler, src_pad)
    out = _gat(sb, xe_gat, elsrc, dst_sp, eler, gw_pad, gb_pad, wr_pad)

    return out[0, 0:1] + jnp.float32(N) * readout_b


# speedup vs baseline: 1.3094x; 1.3094x over previous
import functools

import jax
import jax.numpy as jnp
from jax.experimental import pallas as pl
from jax.experimental.pallas import tpu as pltpu
from jax.experimental.pallas import tpu_sc as plsc

N = 10000
E = 160000
H = 500
HP = 512
G4 = 4 * HP
HEADS = 5
ZP = HEADS * HP
NEG_SLOPE = 0.2

BLK = 256
NPAD = 10240
NBLK = NPAD // BLK
EPAD = E + BLK
CC = 1024
CTRL_LEN = ((E + CC - 1) // CC) * CC
GW = 128

_VMEM = pltpu.MemorySpace.VMEM
_HBM = pltpu.MemorySpace.HBM
_SMEM = pltpu.MemorySpace.SMEM


def _sc_mesh():
    return plsc.VectorSubcoreMesh(core_axis_name="core",
                                  subcore_axis_name="subcore")


def _sc_gather(table, idx):
    orig_dtype = table.dtype
    if orig_dtype == jnp.bfloat16:
        r, w = table.shape
        table = jax.lax.bitcast_convert_type(
            table.reshape(r, w // 2, 2), jnp.uint32)
    width = table.shape[1]
    factor = width // 128
    if factor > 1:
        table = table.reshape(table.shape[0] * factor, 128)
        idx = (idx[:, None] * factor
               + jnp.arange(factor, dtype=jnp.int32)[None, :]).reshape(-1)
        width = 128
    rows = idx.shape[0]
    idx2 = idx.reshape(1, rows)

    @functools.partial(
        pl.kernel,
        out_type=jax.ShapeDtypeStruct((rows, width), table.dtype),
        mesh=_sc_mesh(),
    )
    def k(x_hbm, i_hbm, o_hbm):
        def body(i_vmem, o_vmem):
            pltpu.sync_copy(x_hbm.at[i_vmem.at[0]], o_vmem)

        pltpu.emit_pipeline(
            body,
            grid=(rows // GW,),
            in_specs=[pl.BlockSpec((1, GW), lambda i: (0, i))],
            out_specs=[pl.BlockSpec((GW, width), lambda i: (i, 0))],
            core_axis_name=("core", "subcore"),
            dimension_semantics=(pltpu.PARALLEL,),
        )(i_hbm, o_hbm)

    out = k(table, idx2)
    if factor > 1:
        out = out.reshape(rows // factor, 128 * factor)
    if orig_dtype == jnp.bfloat16:
        out = jax.lax.bitcast_convert_type(out, jnp.bfloat16)
        out = out.reshape(out.shape[0], out.shape[1] * 2)
    return out


def _embed_kernel(tok_ref, emb_ref, o_ref):
    tok = tok_ref[...]
    lanes = jax.lax.broadcasted_iota(jnp.int32, (BLK, 128), 1)
    oh = (tok == lanes).astype(jnp.bfloat16)
    o_ref[...] = jnp.dot(oh, emb_ref[...],
                         preferred_element_type=jnp.float32)


def _embed(tokens_pad, emb_pad):
    return pl.pallas_call(
        _embed_kernel,
        grid=(NBLK,),
        in_specs=[
            pl.BlockSpec((BLK, 1), lambda b: (b, 0)),
            pl.BlockSpec((128, HP), lambda b: (0, 0)),
        ],
        out_specs=pl.BlockSpec((BLK, HP), lambda b: (b, 0)),
        out_shape=jax.ShapeDtypeStruct((NPAD, HP), jnp.float32),
    )(tokens_pad, emb_pad)


def _lstm_kernel(xe_hbm, k_hbm, dmax_ref, wih_ref, whh_ref, b_ref,
                 h_ref, c_ref, xbuf, ksm, dsem, ksem):
    h_ref[...] = jnp.zeros((NPAD, HP), jnp.float32)
    c_ref[...] = jnp.zeros((NPAD, HP), jnp.float32)
    dmax = dmax_ref[0]
    nchunks = (dmax + CC - 1) // CC

    def chunk_body(ci, off):
        cp = pltpu.make_async_copy(k_hbm.at[pl.ds(ci * CC, CC)], ksm, ksem)
        cp.start()
        cp.wait()
        tmax = jnp.minimum(CC, dmax - ci * CC)

        def step(tt, off):
            kt = ksm[tt]
            nblk = (kt + BLK - 1) // BLK

            def blk(j, _):
                base = pl.multiple_of(off + j * BLK, 8)
                jb = pl.multiple_of(j * BLK, BLK)
                cpx = pltpu.make_async_copy(
                    xe_hbm.at[pl.ds(base, BLK), :], xbuf, dsem)
                cpx.start()
                cpx.wait()
                hb = h_ref[pl.ds(jb, BLK), :]
                cb = c_ref[pl.ds(jb, BLK), :]
                gates = (
                    jnp.dot(xbuf[...].astype(jnp.bfloat16), wih_ref[...],
                            preferred_element_type=jnp.float32)
                    + jnp.dot(hb.astype(jnp.bfloat16), whh_ref[...],
                              preferred_element_type=jnp.float32)
                    + b_ref[0:1, :]
                )
                gi = jax.nn.sigmoid(gates[:, 0:HP])
                gf = jax.nn.sigmoid(gates[:, HP:2 * HP])
                gg = jnp.tanh(gates[:, 2 * HP:3 * HP])
                go = jax.nn.sigmoid(gates[:, 3 * HP:4 * HP])
                c_new = gf * cb + gi * gg
                h_new = go * jnp.tanh(c_new)
                rows = jax.lax.broadcasted_iota(jnp.int32, (BLK, 1), 0)
                act = (rows + j * BLK) < kt
                h_ref[pl.ds(jb, BLK), :] = jnp.where(act, h_new, hb)
                c_ref[pl.ds(jb, BLK), :] = jnp.where(act, c_new, cb)
                return 0

            jax.lax.fori_loop(0, nblk, blk, 0)
            return off + kt

        return jax.lax.fori_loop(0, tmax, step, off)

    jax.lax.fori_loop(0, nchunks, chunk_body, jnp.int32(0))


def _lstm(xe, karr, dmax_arr, wih_t, whh_t, bias):
    return pl.pallas_call(
        _lstm_kernel,
        in_specs=[
            pl.BlockSpec(memory_space=_HBM),
            pl.BlockSpec(memory_space=_HBM),
            pl.BlockSpec(memory_space=_SMEM),
            pl.BlockSpec(memory_space=_VMEM),
            pl.BlockSpec(memory_space=_VMEM),
            pl.BlockSpec(memory_space=_VMEM),
        ],
        out_specs=pl.BlockSpec(memory_space=_VMEM),
        out_shape=jax.ShapeDtypeStruct((NPAD, HP), jnp.float32),
        scratch_shapes=[
            pltpu.VMEM((NPAD, HP), jnp.float32),
            pltpu.VMEM((BLK, HP), jnp.float32),
            pltpu.SMEM((CC,), jnp.int32),
            pltpu.SemaphoreType.DMA,
            pltpu.SemaphoreType.DMA,
        ],
        compiler_params=pltpu.CompilerParams(
            vmem_limit_bytes=60 * 1024 * 1024),
    )(xe, karr, dmax_arr, wih_t, whh_t, bias)


def _combine_kernel(x_ref, hn_ref, ws_ref, wn_ref, b_ref, o_ref):
    o_ref[...] = jax.nn.relu(
        jnp.dot(x_ref[...].astype(jnp.bfloat16), ws_ref[...],
                preferred_element_type=jnp.float32)
        + jnp.dot(hn_ref[...].astype(jnp.bfloat16), wn_ref[...],
                  preferred_element_type=jnp.float32)
        + b_ref[0:1, :]
    )


def _combine(x, hn, ws_t, wn_t, bias):
    return pl.pallas_call(
        _combine_kernel,
        grid=(NBLK,),
        in_specs=[
            pl.BlockSpec((BLK, HP), lambda b: (b, 0)),
            pl.BlockSpec((BLK, HP), lambda b: (b, 0)),
            pl.BlockSpec((HP, HP), lambda b: (0, 0)),
            pl.BlockSpec((HP, HP), lambda b: (0, 0)),
            pl.BlockSpec((8, HP), lambda b: (0, 0)),
        ],
        out_specs=pl.BlockSpec((BLK, HP), lambda b: (b, 0)),
        out_shape=jax.ShapeDtypeStruct((NPAD, HP), jnp.float32),
    )(x, hn, ws_t, wn_t, bias)


def _eler_kernel(x_ref, w_ref, o_ref):
    o_ref[...] = jnp.dot(x_ref[...].astype(jnp.bfloat16), w_ref[...],
                         preferred_element_type=jnp.float32)


def _eler(x, w_eler):
    return pl.pallas_call(
        _eler_kernel,
        grid=(NBLK,),
        in_specs=[
            pl.BlockSpec((BLK, HP), lambda b: (b, 0)),
            pl.BlockSpec((HP, 128), lambda b: (0, 0)),
        ],
        out_specs=pl.BlockSpec((BLK, 128), lambda b: (b, 0)),
        out_shape=jax.ShapeDtypeStruct((NPAD, 128), jnp.float32),
    )(x, w_eler)


def _gat_kernel(sb_ref, xe_hbm, els_hbm, dst_hbm, er_ref, gw_ref, gb_ref,
                wr_ref, o_ref, acc_ref, xebuf, elbuf, dstbuf, sem1, sem2,
                sem3):
    b = pl.program_id(0)

    @pl.when(b == 0)
    def _():
        o_ref[...] = jnp.zeros((8, 128), jnp.float32)

    e_begin = sb_ref[b]
    e_end = sb_ref[b + 1]
    nchunks = (e_end - e_begin + BLK - 1) // BLK
    base_node = b * BLK
    lanes = jax.lax.broadcasted_iota(jnp.int32, (BLK, BLK), 1)

    er_rows = []
    for h in range(HEADS):
        col = er_ref[:, 16 + h:16 + h + 1]
        er_rows.append(jnp.transpose(col, (1, 0)))

    def load_meta(k):
        eb = pl.multiple_of(e_begin + k * BLK, 8)
        cp1 = pltpu.make_async_copy(
            els_hbm.at[pl.ds(eb, BLK), :], elbuf, sem1)
        cp2 = pltpu.make_async_copy(
            dst_hbm.at[pl.ds(eb, BLK), :], dstbuf, sem2)
        cp1.start()
        cp2.start()
        cp1.wait()
        cp2.wait()
        valid = (dstbuf[...] - base_node) == lanes
        return valid

    def emat(h, valid):
        e = elbuf[:, h:h + 1] + er_rows[h]
        e = jnp.where(e >= 0, e, NEG_SLOPE * e)
        return e, valid

    def pass1(k, m):
        valid = load_meta(k)
        out = []
        for h in range(HEADS):
            e, v = emat(h, valid)
            e = jnp.where(v, e, -jnp.inf)
            mh = jnp.max(e, axis=0, keepdims=True)
            out.append(jnp.maximum(m[h], mh))
        return tuple(out)

    m0 = tuple(jnp.full((1, BLK), -jnp.inf, jnp.float32)
               for _ in range(HEADS))
    m = jax.lax.fori_loop(0, nchunks, pass1, m0)
    mfin = [jnp.where(jnp.isinf(mh), 0.0, mh) for mh in m]

    acc_ref[...] = jnp.zeros((BLK, ZP), jnp.float32)

    def pass2(k, s):
        valid = load_meta(k)
        cpx = pltpu.make_async_copy(
            xe_hbm.at[pl.ds(pl.multiple_of(e_begin + k * BLK, 8), BLK), :],
            xebuf, sem3)
        cpx.start()
        cpx.wait()
        ze = jnp.dot(xebuf[...], gw_ref[...],
                     preferred_element_type=jnp.float32
                     )
        out = []
        for h in range(HEADS):
            e, v = emat(h, valid)
            w = jnp.where(v, jnp.exp(e - mfin[h]), 0.0)
            sh = jnp.sum(w, axis=0, keepdims=True)
            out.append(s[h] + sh)
            contrib = jax.lax.dot_general(
                w.astype(jnp.bfloat16), ze[:, h * HP:(h + 1) * HP],
                (((0,), (0,)), ((), ())),
                preferred_element_type=jnp.float32)
            acc_ref[:, h * HP:(h + 1) * HP] += contrib
        return tuple(out)

    s0 = tuple(jnp.zeros((1, BLK), jnp.float32) for _ in range(HEADS))
    s = jax.lax.fori_loop(0, nchunks, pass2, s0)

    xm = jnp.zeros((BLK, HP), jnp.float32)
    for h in range(HEADS):
        sT = jnp.transpose(s[h], (1, 0))
        outh = acc_ref[:, h * HP:(h + 1) * HP] / jnp.maximum(sT, 1e-9)
        xm = xm + jax.nn.relu(outh + gb_ref[h:h + 1, :])
    xm = xm * (1.0 / HEADS)
    xm = jnp.where(xm > 0, xm, jnp.exp(xm) - 1.0)
    scores = jnp.dot(xm.astype(jnp.bfloat16), wr_ref[...],
                     preferred_element_type=jnp.float32)
    rows = jax.lax.broadcasted_iota(jnp.int32, (BLK, 128), 0)
    scores = jnp.where(rows + base_node < N, scores, 0.0)
    part = jnp.sum(scores, axis=0, keepdims=True)
    o_ref[...] += jnp.broadcast_to(part, (8, 128))


def _gat(sb, xe_gat, elsrc, dst_sp, er_n, gw_pad, gb_pad, wr_pad):
    return pl.pallas_call(
        _gat_kernel,
        grid=(NBLK,),
        in_specs=[
            pl.BlockSpec(memory_space=_SMEM),
            pl.BlockSpec(memory_space=_HBM),
            pl.BlockSpec(memory_space=_HBM),
            pl.BlockSpec(memory_space=_HBM),
            pl.BlockSpec((BLK, 128), lambda b: (b, 0)),
            pl.BlockSpec((HP, ZP), lambda b: (0, 0)),
            pl.BlockSpec((8, HP), lambda b: (0, 0)),
            pl.BlockSpec((HP, 128), lambda b: (0, 0)),
        ],
        out_specs=pl.BlockSpec((8, 128), lambda b: (0, 0)),
        out_shape=jax.ShapeDtypeStruct((8, 128), jnp.float32),
        scratch_shapes=[
            pltpu.VMEM((BLK, ZP), jnp.float32),
            pltpu.VMEM((BLK, HP), jnp.float32),
            pltpu.VMEM((BLK, 128), jnp.float32),
            pltpu.VMEM((BLK, 1), jnp.int32),
            pltpu.SemaphoreType.DMA,
            pltpu.SemaphoreType.DMA,
            pltpu.SemaphoreType.DMA,
        ],
        compiler_params=pltpu.CompilerParams(
            vmem_limit_bytes=60 * 1024 * 1024),
    )(sb, xe_gat, elsrc, dst_sp, er_n, gw_pad, gb_pad, wr_pad)


def _pad_gate_w(w):
    out = jnp.zeros((HP, G4), jnp.float32)
    for g in range(4):
        out = out.at[:H, g * HP:g * HP + H].set(
            w[g * H:(g + 1) * H, :].T)
    return out


def _pad_gate_b(b):
    out = jnp.zeros((8, G4), jnp.float32)
    for g in range(4):
        out = out.at[:, g * HP:g * HP + H].set(
            jnp.broadcast_to(b[g * H:(g + 1) * H][None, :], (8, H)))
    return out


def kernel(node_tokens, edge_index, emb, sage_Wih, sage_Whh, sage_bih,
           sage_bhh, sage_selfW, sage_selfb, sage_neighW, sage_neighb,
           gat_W, gat_attn_l, gat_attn_r, gat_b, readout_W, readout_b):
    f32 = jnp.float32
    src = edge_index[0].astype(jnp.int32)
    dst = edge_index[1].astype(jnp.int32)
    order = jnp.argsort(dst)
    src_s = src[order]
    dst_s = dst[order]
    deg = jnp.bincount(dst, length=N).astype(jnp.int32)
    starts = (jnp.cumsum(deg) - deg).astype(jnp.int32)
    dmax = jnp.max(deg)

    perm = jnp.argsort(-deg).astype(jnp.int32)
    rank = jnp.zeros((N,), jnp.int32).at[perm].set(
        jnp.arange(N, dtype=jnp.int32))
    cnt = jnp.bincount(deg, length=E + 1)
    le = jnp.cumsum(cnt)
    karr = (N - le)[:E].astype(jnp.int32)
    karr = jnp.concatenate(
        [karr, jnp.zeros((CTRL_LEN - E,), jnp.int32)])
    off = jnp.concatenate(
        [jnp.zeros((1,), jnp.int32),
         jnp.cumsum(karr[:E - 1], dtype=jnp.int32)])
    t_e = jnp.arange(E, dtype=jnp.int32) - starts[dst_s]
    p_e = off[t_e] + rank[dst_s]
    gsrc = jnp.zeros((EPAD,), jnp.int32).at[p_e].set(src_s)
    dmax_arr = jnp.reshape(dmax, (1,)).astype(jnp.int32)

    rank_pad = jnp.concatenate(
        [rank, jnp.zeros((NPAD - N,), jnp.int32)])
    src_pad = jnp.concatenate(
        [src_s, jnp.zeros((EPAD - E,), jnp.int32)])
    dst_sp = jnp.concatenate(
        [dst_s, jnp.full((EPAD - E,), -1, jnp.int32)]).reshape(EPAD, 1)
    sb = jnp.concatenate(
        [starts[::BLK], jnp.full((1,), E, jnp.int32),
         jnp.zeros((7,), jnp.int32)])

    tokens_pad = jnp.concatenate(
        [node_tokens.astype(jnp.int32),
         jnp.zeros((NPAD - N,), jnp.int32)]).reshape(NPAD, 1)
    emb_pad = jnp.zeros((128, HP), f32).at[:119, :H].set(emb).astype(jnp.bfloat16)

    wih_t = [_pad_gate_w(sage_Wih[i]).astype(jnp.bfloat16) for i in range(3)]
    whh_t = [_pad_gate_w(sage_Whh[i]).astype(jnp.bfloat16) for i in range(3)]
    bgate = [_pad_gate_b(sage_bih[i] + sage_bhh[i]) for i in range(3)]
    ws_t = [jnp.zeros((HP, HP), f32).at[:H, :H].set(
        sage_selfW[i].T).astype(jnp.bfloat16) for i in range(3)]
    wn_t = [jnp.zeros((HP, HP), f32).at[:H, :H].set(
        sage_neighW[i].T).astype(jnp.bfloat16) for i in range(3)]
    bcomb = [jnp.zeros((8, HP), f32).at[:, :H].set(
        jnp.broadcast_to((sage_selfb[i] + sage_neighb[i])[None, :], (8, H)))
        for i in range(3)]

    gat_w3 = gat_W.reshape(H, HEADS, H)
    gw_pad = jnp.zeros((HP, ZP), f32)
    for h in range(HEADS):
        gw_pad = gw_pad.at[:H, h * HP:h * HP + H].set(gat_w3[:, h, :])
    gw_pad = gw_pad.astype(jnp.bfloat16)
    w_el = jnp.einsum('ihk,hk->ih', gat_w3, gat_attn_l)
    w_er = jnp.einsum('ihk,hk->ih', gat_w3, gat_attn_r)
    w_eler = jnp.zeros((HP, 128), f32)
    w_eler = w_eler.at[:H, 0:HEADS].set(w_el)
    w_eler = w_eler.at[:H, 16:16 + HEADS].set(w_er)
    w_eler = w_eler.astype(jnp.bfloat16)
    gb_pad = jnp.zeros((8, HP), f32).at[:HEADS, :H].set(
        gat_b.reshape(HEADS, H))
    wr_pad = jnp.zeros((HP, 128), f32).at[:H, 0:1].set(
        readout_W).astype(jnp.bfloat16)

    x = _embed(tokens_pad, emb_pad)
    for i in range(3):
        xe = _sc_gather(x, gsrc)
        h_p = _lstm(xe, karr, dmax_arr, wih_t[i], whh_t[i], bgate[i])
        h_n = _sc_gather(h_p, rank_pad)
        x = _combine(x, h_n, ws_t[i], wn_t[i], bcomb[i])

    eler = _eler(x, w_eler)
    xe_gat = _sc_gather(x, src_pad)
    elsrc = _sc_gather(eler, src_pad)
    out = _gat(sb, xe_gat, elsrc, dst_sp, eler, gw_pad, gb_pad, wr_pad)

    return out[0, 0:1] + jnp.float32(N) * readout_b


# speedup vs baseline: 1.4685x; 1.1215x over previous
import functools

import jax
import jax.numpy as jnp
from jax.experimental import pallas as pl
from jax.experimental.pallas import tpu as pltpu
from jax.experimental.pallas import tpu_sc as plsc

N = 10000
E = 160000
H = 500
HP = 512
G4 = 4 * HP
HEADS = 5
ZP = HEADS * HP
NEG_SLOPE = 0.2

BLK = 256
LBLK = 512
NPAD = 10240
NBLK = NPAD // BLK
EPAD = E + LBLK
CC = 1024
CTRL_LEN = ((E + CC - 1) // CC) * CC
GW = 128

_VMEM = pltpu.MemorySpace.VMEM
_HBM = pltpu.MemorySpace.HBM
_SMEM = pltpu.MemorySpace.SMEM


def _sc_mesh():
    return plsc.VectorSubcoreMesh(core_axis_name="core",
                                  subcore_axis_name="subcore")


def _sc_gather(table, idx):
    orig_dtype = table.dtype
    if orig_dtype == jnp.bfloat16:
        r, w = table.shape
        table = jax.lax.bitcast_convert_type(
            table.reshape(r, w // 2, 2), jnp.uint32)
    width = table.shape[1]
    factor = width // 128
    if factor > 1:
        table = table.reshape(table.shape[0] * factor, 128)
        idx = (idx[:, None] * factor
               + jnp.arange(factor, dtype=jnp.int32)[None, :]).reshape(-1)
        width = 128
    rows = idx.shape[0]
    idx2 = idx.reshape(1, rows)

    @functools.partial(
        pl.kernel,
        out_type=jax.ShapeDtypeStruct((rows, width), table.dtype),
        mesh=_sc_mesh(),
    )
    def k(x_hbm, i_hbm, o_hbm):
        def body(i_vmem, o_vmem):
            pltpu.sync_copy(x_hbm.at[i_vmem.at[0]], o_vmem)

        pltpu.emit_pipeline(
            body,
            grid=(rows // GW,),
            in_specs=[pl.BlockSpec((1, GW), lambda i: (0, i))],
            out_specs=[pl.BlockSpec((GW, width), lambda i: (i, 0))],
            core_axis_name=("core", "subcore"),
            dimension_semantics=(pltpu.PARALLEL,),
        )(i_hbm, o_hbm)

    out = k(table, idx2)
    if factor > 1:
        out = out.reshape(rows // factor, 128 * factor)
    if orig_dtype == jnp.bfloat16:
        out = jax.lax.bitcast_convert_type(out, jnp.bfloat16)
        out = out.reshape(out.shape[0], out.shape[1] * 2)
    return out


def _embed_kernel(tok_ref, emb_ref, o_ref):
    tok = tok_ref[...]
    lanes = jax.lax.broadcasted_iota(jnp.int32, (BLK, 128), 1)
    oh = (tok == lanes).astype(jnp.bfloat16)
    o_ref[...] = jnp.dot(oh, emb_ref[...],
                         preferred_element_type=jnp.float32)


def _embed(tokens_pad, emb_pad):
    return pl.pallas_call(
        _embed_kernel,
        grid=(NBLK,),
        in_specs=[
            pl.BlockSpec((BLK, 1), lambda b: (b, 0)),
            pl.BlockSpec((128, HP), lambda b: (0, 0)),
        ],
        out_specs=pl.BlockSpec((BLK, HP), lambda b: (b, 0)),
        out_shape=jax.ShapeDtypeStruct((NPAD, HP), jnp.float32),
    )(tokens_pad, emb_pad)


def _lstm_kernel(xe_hbm, k_hbm, dmax_ref, wih_ref, whh_ref, b_ref,
                 h_ref, c_ref, xbuf0, xbuf1, ksm, sem0, sem1, ksem):
    h_ref[...] = jnp.zeros((NPAD, HP), jnp.float32)
    c_ref[...] = jnp.zeros((NPAD, HP), jnp.float32)
    dmax = dmax_ref[0]
    nchunks = (dmax + CC - 1) // CC

    def dma(off, j, buf, sem):
        base = pl.multiple_of(off + j * LBLK, 8)
        return pltpu.make_async_copy(
            xe_hbm.at[pl.ds(base, LBLK), :], buf, sem)

    def compute(j, buf, kt):
        jb = pl.multiple_of(j * LBLK, LBLK)
        hb = h_ref[pl.ds(jb, LBLK), :]
        cb = c_ref[pl.ds(jb, LBLK), :]
        gates = (
            jnp.dot(buf[...].astype(jnp.bfloat16), wih_ref[...],
                    preferred_element_type=jnp.float32)
            + jnp.dot(hb.astype(jnp.bfloat16), whh_ref[...],
                      preferred_element_type=jnp.float32)
            + b_ref[0:1, :]
        )
        gi = jax.nn.sigmoid(gates[:, 0:HP])
        gf = jax.nn.sigmoid(gates[:, HP:2 * HP])
        gg = jnp.tanh(gates[:, 2 * HP:3 * HP])
        go = jax.nn.sigmoid(gates[:, 3 * HP:4 * HP])
        c_new = gf * cb + gi * gg
        h_new = go * jnp.tanh(c_new)
        rows = jax.lax.broadcasted_iota(jnp.int32, (LBLK, 1), 0)
        act = (rows + j * LBLK) < kt
        h_ref[pl.ds(jb, LBLK), :] = jnp.where(act, h_new, hb)
        c_ref[pl.ds(jb, LBLK), :] = jnp.where(act, c_new, cb)

    def chunk_body(ci, off):
        cp = pltpu.make_async_copy(k_hbm.at[pl.ds(ci * CC, CC)], ksm, ksem)
        cp.start()
        cp.wait()
        tmax = jnp.minimum(CC, dmax - ci * CC)

        def step(tt, off):
            kt = ksm[tt]
            nblk = (kt + LBLK - 1) // LBLK
            dma(off, 0, xbuf0, sem0).start()

            def pair(jj, _):
                j0 = 2 * jj
                j1 = j0 + 1

                @pl.when(j1 < nblk)
                def _():
                    dma(off, j1, xbuf1, sem1).start()

                dma(off, j0, xbuf0, sem0).wait()
                compute(j0, xbuf0, kt)

                @pl.when(j1 < nblk)
                def _():
                    @pl.when(j1 + 1 < nblk)
                    def _():
                        dma(off, j1 + 1, xbuf0, sem0).start()

                    dma(off, j1, xbuf1, sem1).wait()
                    compute(j1, xbuf1, kt)

                return 0

            jax.lax.fori_loop(0, (nblk + 1) // 2, pair, 0)
            return off + kt

        return jax.lax.fori_loop(0, tmax, step, off)

    jax.lax.fori_loop(0, nchunks, chunk_body, jnp.int32(0))


def _lstm(xe, karr, dmax_arr, wih_t, whh_t, bias):
    return pl.pallas_call(
        _lstm_kernel,
        in_specs=[
            pl.BlockSpec(memory_space=_HBM),
            pl.BlockSpec(memory_space=_HBM),
            pl.BlockSpec(memory_space=_SMEM),
            pl.BlockSpec(memory_space=_VMEM),
            pl.BlockSpec(memory_space=_VMEM),
            pl.BlockSpec(memory_space=_VMEM),
        ],
        out_specs=pl.BlockSpec(memory_space=_VMEM),
        out_shape=jax.ShapeDtypeStruct((NPAD, HP), jnp.float32),
        scratch_shapes=[
            pltpu.VMEM((NPAD, HP), jnp.float32),
            pltpu.VMEM((LBLK, HP), jnp.float32),
            pltpu.VMEM((LBLK, HP), jnp.float32),
            pltpu.SMEM((CC,), jnp.int32),
            pltpu.SemaphoreType.DMA,
            pltpu.SemaphoreType.DMA,
            pltpu.SemaphoreType.DMA,
        ],
        compiler_params=pltpu.CompilerParams(
            vmem_limit_bytes=60 * 1024 * 1024),
    )(xe, karr, dmax_arr, wih_t, whh_t, bias)


def _combine_kernel(x_ref, hn_ref, ws_ref, wn_ref, b_ref, o_ref):
    o_ref[...] = jax.nn.relu(
        jnp.dot(x_ref[...].astype(jnp.bfloat16), ws_ref[...],
                preferred_element_type=jnp.float32)
        + jnp.dot(hn_ref[...].astype(jnp.bfloat16), wn_ref[...],
                  preferred_element_type=jnp.float32)
        + b_ref[0:1, :]
    )


def _combine(x, hn, ws_t, wn_t, bias):
    return pl.pallas_call(
        _combine_kernel,
        grid=(NBLK,),
        in_specs=[
            pl.BlockSpec((BLK, HP), lambda b: (b, 0)),
            pl.BlockSpec((BLK, HP), lambda b: (b, 0)),
            pl.BlockSpec((HP, HP), lambda b: (0, 0)),
            pl.BlockSpec((HP, HP), lambda b: (0, 0)),
            pl.BlockSpec((8, HP), lambda b: (0, 0)),
        ],
        out_specs=pl.BlockSpec((BLK, HP), lambda b: (b, 0)),
        out_shape=jax.ShapeDtypeStruct((NPAD, HP), jnp.float32),
    )(x, hn, ws_t, wn_t, bias)


def _eler_kernel(x_ref, w_ref, o_ref):
    o_ref[...] = jnp.dot(x_ref[...].astype(jnp.bfloat16), w_ref[...],
                         preferred_element_type=jnp.float32)


def _eler(x, w_eler):
    return pl.pallas_call(
        _eler_kernel,
        grid=(NBLK,),
        in_specs=[
            pl.BlockSpec((BLK, HP), lambda b: (b, 0)),
            pl.BlockSpec((HP, 128), lambda b: (0, 0)),
        ],
        out_specs=pl.BlockSpec((BLK, 128), lambda b: (b, 0)),
        out_shape=jax.ShapeDtypeStruct((NPAD, 128), jnp.float32),
    )(x, w_eler)


def _gat_kernel(sb_ref, xe_hbm, els_hbm, dst_hbm, er_ref, gw_ref, gb_ref,
                wr_ref, o_ref, acc_ref, xebuf, elbuf, dstbuf, sem1, sem2,
                sem3):
    b = pl.program_id(0)

    @pl.when(b == 0)
    def _():
        o_ref[...] = jnp.zeros((8, 128), jnp.float32)

    e_begin = sb_ref[b]
    e_end = sb_ref[b + 1]
    nchunks = (e_end - e_begin + BLK - 1) // BLK
    base_node = b * BLK
    lanes = jax.lax.broadcasted_iota(jnp.int32, (BLK, BLK), 1)

    er_rows = []
    for h in range(HEADS):
        col = er_ref[:, 16 + h:16 + h + 1]
        er_rows.append(jnp.transpose(col, (1, 0)))

    def load_meta(k):
        eb = pl.multiple_of(e_begin + k * BLK, 8)
        cp1 = pltpu.make_async_copy(
            els_hbm.at[pl.ds(eb, BLK), :], elbuf, sem1)
        cp2 = pltpu.make_async_copy(
            dst_hbm.at[pl.ds(eb, BLK), :], dstbuf, sem2)
        cp1.start()
        cp2.start()
        cp1.wait()
        cp2.wait()
        valid = (dstbuf[...] - base_node) == lanes
        return valid

    def emat(h, valid):
        e = elbuf[:, h:h + 1] + er_rows[h]
        e = jnp.where(e >= 0, e, NEG_SLOPE * e)
        return e, valid

    def pass1(k, m):
        valid = load_meta(k)
        out = []
        for h in range(HEADS):
            e, v = emat(h, valid)
            e = jnp.where(v, e, -jnp.inf)
            mh = jnp.max(e, axis=0, keepdims=True)
            out.append(jnp.maximum(m[h], mh))
        return tuple(out)

    m0 = tuple(jnp.full((1, BLK), -jnp.inf, jnp.float32)
               for _ in range(HEADS))
    m = jax.lax.fori_loop(0, nchunks, pass1, m0)
    mfin = [jnp.where(jnp.isinf(mh), 0.0, mh) for mh in m]

    acc_ref[...] = jnp.zeros((BLK, ZP), jnp.float32)

    def pass2(k, s):
        valid = load_meta(k)
        cpx = pltpu.make_async_copy(
            xe_hbm.at[pl.ds(pl.multiple_of(e_begin + k * BLK, 8), BLK), :],
            xebuf, sem3)
        cpx.start()
        cpx.wait()
        ze = jnp.dot(xebuf[...], gw_ref[...],
                     preferred_element_type=jnp.float32
                     )
        out = []
        for h in range(HEADS):
            e, v = emat(h, valid)
            w = jnp.where(v, jnp.exp(e - mfin[h]), 0.0)
            sh = jnp.sum(w, axis=0, keepdims=True)
            out.append(s[h] + sh)
            contrib = jax.lax.dot_general(
                w.astype(jnp.bfloat16), ze[:, h * HP:(h + 1) * HP],
                (((0,), (0,)), ((), ())),
                preferred_element_type=jnp.float32)
            acc_ref[:, h * HP:(h + 1) * HP] += contrib
        return tuple(out)

    s0 = tuple(jnp.zeros((1, BLK), jnp.float32) for _ in range(HEADS))
    s = jax.lax.fori_loop(0, nchunks, pass2, s0)

    xm = jnp.zeros((BLK, HP), jnp.float32)
    for h in range(HEADS):
        sT = jnp.transpose(s[h], (1, 0))
        outh = acc_ref[:, h * HP:(h + 1) * HP] / jnp.maximum(sT, 1e-9)
        xm = xm + jax.nn.relu(outh + gb_ref[h:h + 1, :])
    xm = xm * (1.0 / HEADS)
    xm = jnp.where(xm > 0, xm, jnp.exp(xm) - 1.0)
    scores = jnp.dot(xm.astype(jnp.bfloat16), wr_ref[...],
                     preferred_element_type=jnp.float32)
    rows = jax.lax.broadcasted_iota(jnp.int32, (BLK, 128), 0)
    scores = jnp.where(rows + base_node < N, scores, 0.0)
    part = jnp.sum(scores, axis=0, keepdims=True)
    o_ref[...] += jnp.broadcast_to(part, (8, 128))


def _gat(sb, xe_gat, elsrc, dst_sp, er_n, gw_pad, gb_pad, wr_pad):
    return pl.pallas_call(
        _gat_kernel,
        grid=(NBLK,),
        in_specs=[
            pl.BlockSpec(memory_space=_SMEM),
            pl.BlockSpec(memory_space=_HBM),
            pl.BlockSpec(memory_space=_HBM),
            pl.BlockSpec(memory_space=_HBM),
            pl.BlockSpec((BLK, 128), lambda b: (b, 0)),
            pl.BlockSpec((HP, ZP), lambda b: (0, 0)),
            pl.BlockSpec((8, HP), lambda b: (0, 0)),
            pl.BlockSpec((HP, 128), lambda b: (0, 0)),
        ],
        out_specs=pl.BlockSpec((8, 128), lambda b: (0, 0)),
        out_shape=jax.ShapeDtypeStruct((8, 128), jnp.float32),
        scratch_shapes=[
            pltpu.VMEM((BLK, ZP), jnp.float32),
            pltpu.VMEM((BLK, HP), jnp.float32),
            pltpu.VMEM((BLK, 128), jnp.float32),
            pltpu.VMEM((BLK, 1), jnp.int32),
            pltpu.SemaphoreType.DMA,
            pltpu.SemaphoreType.DMA,
            pltpu.SemaphoreType.DMA,
        ],
        compiler_params=pltpu.CompilerParams(
            vmem_limit_bytes=60 * 1024 * 1024),
    )(sb, xe_gat, elsrc, dst_sp, er_n, gw_pad, gb_pad, wr_pad)


def _pad_gate_w(w):
    out = jnp.zeros((HP, G4), jnp.float32)
    for g in range(4):
        out = out.at[:H, g * HP:g * HP + H].set(
            w[g * H:(g + 1) * H, :].T)
    return out


def _pad_gate_b(b):
    out = jnp.zeros((8, G4), jnp.float32)
    for g in range(4):
        out = out.at[:, g * HP:g * HP + H].set(
            jnp.broadcast_to(b[g * H:(g + 1) * H][None, :], (8, H)))
    return out


def kernel(node_tokens, edge_index, emb, sage_Wih, sage_Whh, sage_bih,
           sage_bhh, sage_selfW, sage_selfb, sage_neighW, sage_neighb,
           gat_W, gat_attn_l, gat_attn_r, gat_b, readout_W, readout_b):
    f32 = jnp.float32
    src = edge_index[0].astype(jnp.int32)
    dst = edge_index[1].astype(jnp.int32)
    order = jnp.argsort(dst)
    src_s = src[order]
    dst_s = dst[order]
    deg = jnp.bincount(dst, length=N).astype(jnp.int32)
    starts = (jnp.cumsum(deg) - deg).astype(jnp.int32)
    dmax = jnp.max(deg)

    perm = jnp.argsort(-deg).astype(jnp.int32)
    rank = jnp.zeros((N,), jnp.int32).at[perm].set(
        jnp.arange(N, dtype=jnp.int32))
    cnt = jnp.bincount(deg, length=E + 1)
    le = jnp.cumsum(cnt)
    karr = (N - le)[:E].astype(jnp.int32)
    karr = jnp.concatenate(
        [karr, jnp.zeros((CTRL_LEN - E,), jnp.int32)])
    off = jnp.concatenate(
        [jnp.zeros((1,), jnp.int32),
         jnp.cumsum(karr[:E - 1], dtype=jnp.int32)])
    t_e = jnp.arange(E, dtype=jnp.int32) - starts[dst_s]
    p_e = off[t_e] + rank[dst_s]
    gsrc = jnp.zeros((EPAD,), jnp.int32).at[p_e].set(src_s)
    dmax_arr = jnp.reshape(dmax, (1,)).astype(jnp.int32)

    rank_pad = jnp.concatenate(
        [rank, jnp.zeros((NPAD - N,), jnp.int32)])
    src_pad = jnp.concatenate(
        [src_s, jnp.zeros((EPAD - E,), jnp.int32)])
    dst_sp = jnp.concatenate(
        [dst_s, jnp.full((EPAD - E,), -1, jnp.int32)]).reshape(EPAD, 1)
    sb = jnp.concatenate(
        [starts[::BLK], jnp.full((1,), E, jnp.int32),
         jnp.zeros((7,), jnp.int32)])

    tokens_pad = jnp.concatenate(
        [node_tokens.astype(jnp.int32),
         jnp.zeros((NPAD - N,), jnp.int32)]).reshape(NPAD, 1)
    emb_pad = jnp.zeros((128, HP), f32).at[:119, :H].set(emb).astype(jnp.bfloat16)

    wih_t = [_pad_gate_w(sage_Wih[i]).astype(jnp.bfloat16) for i in range(3)]
    whh_t = [_pad_gate_w(sage_Whh[i]).astype(jnp.bfloat16) for i in range(3)]
    bgate = [_pad_gate_b(sage_bih[i] + sage_bhh[i]) for i in range(3)]
    ws_t = [jnp.zeros((HP, HP), f32).at[:H, :H].set(
        sage_selfW[i].T).astype(jnp.bfloat16) for i in range(3)]
    wn_t = [jnp.zeros((HP, HP), f32).at[:H, :H].set(
        sage_neighW[i].T).astype(jnp.bfloat16) for i in range(3)]
    bcomb = [jnp.zeros((8, HP), f32).at[:, :H].set(
        jnp.broadcast_to((sage_selfb[i] + sage_neighb[i])[None, :], (8, H)))
        for i in range(3)]

    gat_w3 = gat_W.reshape(H, HEADS, H)
    gw_pad = jnp.zeros((HP, ZP), f32)
    for h in range(HEADS):
        gw_pad = gw_pad.at[:H, h * HP:h * HP + H].set(gat_w3[:, h, :])
    gw_pad = gw_pad.astype(jnp.bfloat16)
    w_el = jnp.einsum('ihk,hk->ih', gat_w3, gat_attn_l)
    w_er = jnp.einsum('ihk,hk->ih', gat_w3, gat_attn_r)
    w_eler = jnp.zeros((HP, 128), f32)
    w_eler = w_eler.at[:H, 0:HEADS].set(w_el)
    w_eler = w_eler.at[:H, 16:16 + HEADS].set(w_er)
    w_eler = w_eler.astype(jnp.bfloat16)
    gb_pad = jnp.zeros((8, HP), f32).at[:HEADS, :H].set(
        gat_b.reshape(HEADS, H))
    wr_pad = jnp.zeros((HP, 128), f32).at[:H, 0:1].set(
        readout_W).astype(jnp.bfloat16)

    x = _embed(tokens_pad, emb_pad)
    for i in range(3):
        xe = _sc_gather(x, gsrc)
        h_p = _lstm(xe, karr, dmax_arr, wih_t[i], whh_t[i], bgate[i])
        h_n = _sc_gather(h_p, rank_pad)
        x = _combine(x, h_n, ws_t[i], wn_t[i], bcomb[i])

    eler = _eler(x, w_eler)
    xe_gat = _sc_gather(x, src_pad)
    elsrc = _sc_gather(eler, src_pad)
    out = _gat(sb, xe_gat, elsrc, dst_sp, eler, gw_pad, gb_pad, wr_pad)

    return out[0, 0:1] + jnp.float32(N) * readout_b
